# Initial kernel scaffold; baseline (speedup 1.0000x reference)
import jax
import jax.numpy as jnp
from jax.experimental import pallas as pl


def kernel(input, train_X, train_Y):
    raise NotImplementedError("write your pallas kernel here")


# speedup vs baseline: 399.9292x; 399.9292x over previous
import jax
import jax.numpy as jnp
from jax.experimental import pallas as pl
from jax.experimental.pallas import tpu as pltpu

_NCLS = 1000
_CPAD = 1024
_RADIUS = 4.0
_L2_THRESH = _RADIUS * _RADIUS + 0.05


def _bincount_body(y_ref, mf_ref, counts_ref):
    step = pl.program_id(0)
    nsteps = pl.num_programs(0)

    @pl.when(step == 0)
    def _init():
        counts_ref[...] = jnp.zeros_like(counts_ref)

    rows = y_ref.shape[0] // 128
    for j in range(rows):
        ys = y_ref[j * 128:(j + 1) * 128, :]
        ii = jax.lax.broadcasted_iota(jnp.int32, (128, _CPAD), 1)
        oh = (ys == ii).astype(jnp.float32)
        counts_ref[j:j + 1, :] += jnp.sum(oh, axis=0, keepdims=True)

    @pl.when(step == nsteps - 1)
    def _fin():
        total = jnp.sum(counts_ref[...], axis=0, keepdims=True)
        lane = jax.lax.broadcasted_iota(jnp.int32, (1, _CPAD), 1)
        masked = jnp.where(lane < _NCLS, total, -1.0)
        m = jnp.max(masked, axis=1, keepdims=True)
        sel = jnp.where(masked == m, lane, jnp.int32(2 ** 30))
        mf_ref[0, 0] = jnp.min(sel)


def _main_body(mf_ref, q_ref, xt_ref, y_ref, votes_ref, nbr_ref, preds_ref):
    ki = pl.program_id(1)
    nk = pl.num_programs(1)

    @pl.when(ki == 0)
    def _init():
        votes_ref[...] = jnp.zeros_like(votes_ref)
        nbr_ref[...] = jnp.zeros_like(nbr_ref)

    q = q_ref[...]
    xt = xt_ref[...]
    dot = jax.lax.dot_general(q, xt, (((1,), (0,)), ((), ())),
                              preferred_element_type=jnp.float32)
    nx = jnp.sum(xt * xt, axis=0, keepdims=True)
    nq = jnp.sum(q * q, axis=1, keepdims=True)
    l2sq = (nq + nx) - 2.0 * dot
    anyc = jnp.any(l2sq <= _L2_THRESH)

    @pl.when(anyc)
    def _exact():
        qt, d_dim = q.shape
        kt = xt.shape[1]
        acc = jnp.zeros((qt, kt), jnp.float32)
        for d in range(d_dim):
            acc = acc + jnp.abs(q[:, d:d + 1] - xt[d:d + 1, :])
        within = acc <= _RADIUS
        w = jnp.where(within, 1.0 / jnp.maximum(acc, 1e-12), 0.0)
        cnt = jnp.sum(within.astype(jnp.float32), axis=1, keepdims=True)
        nbr_ref[...] += jnp.broadcast_to(cnt, nbr_ref.shape)
        for j in range(kt // 256):
            ys = y_ref[j * 256:(j + 1) * 256, :]
            ii = jax.lax.broadcasted_iota(jnp.int32, (256, _CPAD), 1)
            oh = (ys == ii).astype(jnp.float32)
            votes_ref[...] += jax.lax.dot_general(
                w[:, j * 256:(j + 1) * 256], oh, (((1,), (0,)), ((), ())),
                preferred_element_type=jnp.float32)

    @pl.when(ki == nk - 1)
    def _fin():
        votes = votes_ref[...]
        m = jnp.max(votes, axis=1, keepdims=True)
        ii = jax.lax.broadcasted_iota(jnp.int32, votes.shape, 1)
        am = jnp.min(jnp.where(votes == m, ii, jnp.int32(2 ** 30)),
                     axis=1, keepdims=True)
        hasn = nbr_ref[...][:, 0:1] > 0.0
        pred = jnp.where(hasn, am, mf_ref[0, 0])
        preds_ref[...] = jnp.broadcast_to(pred, preds_ref.shape)


def kernel(input, train_X, train_Y):
    q_n, d_dim = input.shape
    k_n = train_X.shape[0]
    qt = 256 if q_n % 256 == 0 else q_n
    kt = 2048
    nk = -(-k_n // kt)
    k_pad = nk * kt
    nq = q_n // qt

    xt = jnp.concatenate(
        [train_X.T, jnp.full((d_dim, k_pad - k_n), 1e6, jnp.float32)], axis=1)
    y_col = jnp.concatenate(
        [train_Y, jnp.full((k_pad - k_n,), _CPAD - 1, jnp.int32)]
    ).reshape(k_pad, 1)

    mf = pl.pallas_call(
        _bincount_body,
        grid=(k_pad // 1024,),
        in_specs=[pl.BlockSpec((1024, 1), lambda i: (i, 0))],
        out_specs=pl.BlockSpec(memory_space=pltpu.SMEM),
        out_shape=jax.ShapeDtypeStruct((1, 1), jnp.int32),
        scratch_shapes=[pltpu.VMEM((8, _CPAD), jnp.float32)],
    )(y_col)

    votes, nbrs, preds = pl.pallas_call(
        _main_body,
        grid=(nq, nk),
        in_specs=[
            pl.BlockSpec(memory_space=pltpu.SMEM),
            pl.BlockSpec((qt, d_dim), lambda qi, ki: (qi, 0)),
            pl.BlockSpec((d_dim, kt), lambda qi, ki: (0, ki)),
            pl.BlockSpec((kt, 1), lambda qi, ki: (ki, 0)),
        ],
        out_specs=[
            pl.BlockSpec((qt, _CPAD), lambda qi, ki: (qi, 0)),
            pl.BlockSpec((qt, 128), lambda qi, ki: (qi, 0)),
            pl.BlockSpec((qt, 128), lambda qi, ki: (qi, 0)),
        ],
        out_shape=[
            jax.ShapeDtypeStruct((q_n, _CPAD), jnp.float32),
            jax.ShapeDtypeStruct((q_n, 128), jnp.float32),
            jax.ShapeDtypeStruct((q_n, 128), jnp.int32),
        ],
    )(mf, input, xt, y_col)
    return preds[:, 0]
